# Initial kernel scaffold; baseline (speedup 1.0000x reference)
import jax
import jax.numpy as jnp
from jax.experimental import pallas as pl


def kernel(x, edge_index, edge_attr, W_ih, W_hh, b_ih, b_hh, gcn_W, gcn_b, out_W, out_b):
    raise NotImplementedError("write your pallas kernel here")


# speedup vs baseline: 4.1012x; 4.1012x over previous
import functools

import jax
import jax.numpy as jnp
from jax import lax
from jax.experimental import pallas as pl
from jax.experimental.pallas import tpu as pltpu
from jax.experimental.pallas import tpu_sc as plsc

_NC = 2
_NS = 16
_CHUNK = 128


def _lstm_tc(x2, wihT, whhT, b2, gcnwT, n_nodes, t_steps, d_in, hid, bn):
    nb = n_nodes // bn

    def body(x_ref, wih_ref, whh_ref, b_ref, gcnw_ref, h_ref, xw_ref):
        h = jnp.zeros((bn, hid), jnp.float32)
        c = jnp.zeros((bn, hid), jnp.float32)
        wih = wih_ref[...]
        whh = whh_ref[...]
        b = b_ref[...]
        for t in range(t_steps):
            xt = x_ref[:, t * d_in:(t + 1) * d_in]
            gates = (jnp.dot(xt, wih, preferred_element_type=jnp.float32)
                     + jnp.dot(h, whh, preferred_element_type=jnp.float32)
                     + b)
            i = jax.nn.sigmoid(gates[:, :hid])
            f = jax.nn.sigmoid(gates[:, hid:2 * hid])
            g = jnp.tanh(gates[:, 2 * hid:3 * hid])
            o = jax.nn.sigmoid(gates[:, 3 * hid:])
            c = f * c + i * g
            h = o * jnp.tanh(c)
        h_ref[...] = h
        xw_ref[...] = jnp.dot(h, gcnw_ref[...], preferred_element_type=jnp.float32)

    return pl.pallas_call(
        body,
        grid=(nb,),
        in_specs=[
            pl.BlockSpec((bn, t_steps * d_in), lambda i: (i, 0)),
            pl.BlockSpec((d_in, 4 * hid), lambda i: (0, 0)),
            pl.BlockSpec((hid, 4 * hid), lambda i: (0, 0)),
            pl.BlockSpec((1, 4 * hid), lambda i: (0, 0)),
            pl.BlockSpec((hid, hid), lambda i: (0, 0)),
        ],
        out_specs=[
            pl.BlockSpec((bn, hid), lambda i: (i, 0)),
            pl.BlockSpec((bn, hid), lambda i: (i, 0)),
        ],
        out_shape=[
            jax.ShapeDtypeStruct((n_nodes, hid), jnp.float32),
            jax.ShapeDtypeStruct((n_nodes, hid), jnp.float32),
        ],
        compiler_params=pltpu.CompilerParams(
            dimension_semantics=("arbitrary",)),
    )(x2, wihT, whhT, b2, gcnwT)


def _sc_deg(dst3, w3, n_nodes):
    nchunk = dst3.shape[1]
    base = (n_nodes // (8 * _NS)) * 8
    last = n_nodes - (_NS - 1) * base
    mesh = plsc.VectorSubcoreMesh(
        core_axis_name="c", subcore_axis_name="s",
        num_cores=_NC, num_subcores=_NS)

    @functools.partial(
        pl.kernel,
        out_type=pltpu.HBM((_NC, n_nodes, 128), jnp.float32),
        mesh=mesh,
        scratch_types=[
            pltpu.VMEM((nchunk, _CHUNK), jnp.int32),
            pltpu.VMEM((nchunk, _CHUNK), jnp.float32),
            pltpu.VMEM((_CHUNK, 128), jnp.float32),
            pltpu.VMEM((16, 128), jnp.float32),
            pltpu.VMEM_SHARED((n_nodes, 128), jnp.float32),
        ],
    )
    def k(dst_hbm, w_hbm, out_hbm, dstv, wv, rows, zbuf, deg_s):
        cid = lax.axis_index("c")
        sid = lax.axis_index("s")
        wid = cid * _NS + sid
        pltpu.sync_copy(dst_hbm.at[wid], dstv)
        pltpu.sync_copy(w_hbm.at[wid], wv)

        zv = jnp.zeros((16,), jnp.float32)
        for r in range(16):
            for q in range(8):
                zbuf[r, pl.ds(q * 16, 16)] = zv
        def zrows(e, carry):
            for q in range(1, 8):
                rows[e, pl.ds(q * 16, 16)] = zv
            return carry
        lax.fori_loop(0, _CHUNK, zrows, 0)
        nz = jnp.where(sid == _NS - 1, last // 16, base // 16)

        def zero_chunk(i, carry):
            pltpu.sync_copy(zbuf, deg_s.at[pl.ds(sid * base + i * 16, 16)])
            return carry
        lax.fori_loop(0, nz, zero_chunk, 0)
        plsc.subcore_barrier()

        def chunk_body(j, carry):
            def fill(g, c2):
                wvec = wv[j, pl.ds(g * 16, 16)]
                for l in range(16):
                    rows[g * 16 + l, pl.ds(0, 16)] = jnp.full(
                        (16,), wvec[l], jnp.float32)
                return c2
            lax.fori_loop(0, _CHUNK // 16, fill, 0)
            pltpu.sync_copy(rows, deg_s.at[dstv.at[j]], add=True)
            return carry
        lax.fori_loop(0, nchunk, chunk_body, 0)
        plsc.subcore_barrier()

        @pl.when(sid < _NS - 1)
        def _():
            pltpu.sync_copy(deg_s.at[pl.ds(sid * base, base)],
                            out_hbm.at[cid, pl.ds(sid * base, base)])

        @pl.when(sid == _NS - 1)
        def _():
            pltpu.sync_copy(deg_s.at[pl.ds((_NS - 1) * base, last)],
                            out_hbm.at[cid, pl.ds((_NS - 1) * base, last)])

    return k(dst3, w3)


def _disy_tc(degp, xw, n_nodes, hid, bnb):
    nb = n_nodes // bnb

    def body(dp_ref, xw_ref, dis_ref, y_ref):
        dp = dp_ref[:, :, :16]
        deg = jnp.sum(dp, axis=(0, 2)) * (1.0 / 16.0) + 1.0
        dis = jnp.where(deg > 0, lax.rsqrt(deg), 0.0)
        dis_ref[...] = dis[:, None]
        y_ref[:, :hid] = xw_ref[...] * dis[:, None]
        y_ref[:, hid:] = jnp.zeros((y_ref.shape[0], 128 - hid), jnp.float32)

    return pl.pallas_call(
        body,
        grid=(nb,),
        in_specs=[
            pl.BlockSpec((_NC, bnb, 128), lambda i: (0, i, 0)),
            pl.BlockSpec((bnb, hid), lambda i: (i, 0)),
        ],
        out_specs=[
            pl.BlockSpec((bnb, 1), lambda i: (i, 0)),
            pl.BlockSpec((bnb, 128), lambda i: (i, 0)),
        ],
        out_shape=[
            jax.ShapeDtypeStruct((n_nodes, 1), jnp.float32),
            jax.ShapeDtypeStruct((n_nodes, 128), jnp.float32),
        ],
    )(degp, xw)


def _sc_msg(src3, dst3, w3, y, n_nodes, hid):
    nchunk = src3.shape[1]
    base = (n_nodes // (8 * _NS)) * 8
    last = n_nodes - (_NS - 1) * base
    mesh = plsc.VectorSubcoreMesh(
        core_axis_name="c", subcore_axis_name="s",
        num_cores=_NC, num_subcores=_NS)

    @functools.partial(
        pl.kernel,
        out_type=pltpu.HBM((_NC, n_nodes, 128), jnp.float32),
        mesh=mesh,
        scratch_types=[
            pltpu.VMEM((nchunk, _CHUNK), jnp.int32),
            pltpu.VMEM((nchunk, _CHUNK), jnp.int32),
            pltpu.VMEM((nchunk, _CHUNK), jnp.float32),
            pltpu.VMEM((_CHUNK, 128), jnp.float32),
            pltpu.VMEM((16, 128), jnp.float32),
            pltpu.VMEM_SHARED((n_nodes, 128), jnp.float32),
            pltpu.SemaphoreType.DMA,
        ],
    )
    def k(src_hbm, dst_hbm, w_hbm, y_hbm, out_hbm,
          srcv, dstv, wv, rows, zbuf, acc_s, sem):
        cid = lax.axis_index("c")
        sid = lax.axis_index("s")
        wid = cid * _NS + sid
        pltpu.sync_copy(src_hbm.at[wid], srcv)
        pltpu.sync_copy(dst_hbm.at[wid], dstv)
        pltpu.sync_copy(w_hbm.at[wid], wv)

        zv = jnp.zeros((16,), jnp.float32)
        for r in range(16):
            for q in range(8):
                zbuf[r, pl.ds(q * 16, 16)] = zv
        nz = jnp.where(sid == _NS - 1, last // 16, base // 16)

        def zero_chunk(i, carry):
            pltpu.sync_copy(zbuf, acc_s.at[pl.ds(sid * base + i * 16, 16)])
            return carry
        lax.fori_loop(0, nz, zero_chunk, 0)
        plsc.subcore_barrier()

        def chunk_body(j, carry):
            pltpu.async_copy(y_hbm.at[srcv.at[j]], rows, sem).wait()

            def scale(g, c2):
                wvec = wv[j, pl.ds(g * 16, 16)]
                for l in range(16):
                    e = g * 16 + l
                    w = wvec[l]
                    rows[e, pl.ds(0, 16)] = rows[e, pl.ds(0, 16)] * w
                    rows[e, pl.ds(16, 16)] = rows[e, pl.ds(16, 16)] * w
                return c2
            lax.fori_loop(0, _CHUNK // 16, scale, 0)
            pltpu.sync_copy(rows, acc_s.at[dstv.at[j]], add=True)
            return carry
        lax.fori_loop(0, nchunk, chunk_body, 0)
        plsc.subcore_barrier()

        @pl.when(sid < _NS - 1)
        def _():
            pltpu.sync_copy(acc_s.at[pl.ds(sid * base, base)],
                            out_hbm.at[cid, pl.ds(sid * base, base)])

        @pl.when(sid == _NS - 1)
        def _():
            pltpu.sync_copy(acc_s.at[pl.ds((_NS - 1) * base, last)],
                            out_hbm.at[cid, pl.ds((_NS - 1) * base, last)])

    return k(src3, dst3, w3, y)


def _head_tc(accp, dis, xw, h, gcnb2, wg, wh, ob2, n_nodes, hid, bnb):
    nb = n_nodes // bnb

    def body(a_ref, dis_ref, xw_ref, h_ref, gcnb_ref, wg_ref, wh_ref, ob_ref,
             o_ref):
        dis_c = dis_ref[...]
        acc = a_ref[0, :, :hid] + a_ref[1, :, :hid]
        gcn = (acc * dis_c
               + xw_ref[...] * (dis_c * dis_c) + gcnb_ref[...])
        zg = jnp.where(gcn > 0, gcn, jnp.exp(jnp.minimum(gcn, 0.0)) - 1.0)
        hh = h_ref[...]
        zh = jnp.where(hh > 0, hh, jnp.exp(jnp.minimum(hh, 0.0)) - 1.0)
        s = (jnp.dot(zg, wg_ref[...], preferred_element_type=jnp.float32)
             + jnp.dot(zh, wh_ref[...], preferred_element_type=jnp.float32)
             + ob_ref[...])
        o_ref[...] = jax.nn.sigmoid(s)

    return pl.pallas_call(
        body,
        grid=(nb,),
        in_specs=[
            pl.BlockSpec((_NC, bnb, 128), lambda i: (0, i, 0)),
            pl.BlockSpec((bnb, 1), lambda i: (i, 0)),
            pl.BlockSpec((bnb, hid), lambda i: (i, 0)),
            pl.BlockSpec((bnb, hid), lambda i: (i, 0)),
            pl.BlockSpec((1, hid), lambda i: (0, 0)),
            pl.BlockSpec((hid, 1), lambda i: (0, 0)),
            pl.BlockSpec((hid, 1), lambda i: (0, 0)),
            pl.BlockSpec((1, 1), lambda i: (0, 0)),
        ],
        out_specs=pl.BlockSpec((bnb, 1), lambda i: (i, 0)),
        out_shape=jax.ShapeDtypeStruct((n_nodes, 1), jnp.float32),
    )(accp, dis, xw, h, gcnb2, wg, wh, ob2)


def kernel(x, edge_index, edge_attr, W_ih, W_hh, b_ih, b_hh,
           gcn_W, gcn_b, out_W, out_b):
    n_nodes, t_steps, d_in = x.shape
    hid = W_hh.shape[1]
    e_edges = edge_attr.shape[0]
    nw = _NC * _NS

    grp = nw * _CHUNK
    epad = ((e_edges + grp - 1) // grp) * grp
    pad = epad - e_edges
    src = edge_index[0].astype(jnp.int32)
    dst = edge_index[1].astype(jnp.int32)
    w = edge_attr
    if pad:
        src = jnp.concatenate([src, jnp.zeros((pad,), jnp.int32)])
        dst = jnp.concatenate([dst, jnp.zeros((pad,), jnp.int32)])
        w = jnp.concatenate([w, jnp.zeros((pad,), w.dtype)])
    nchunk = epad // grp
    src3 = src.reshape(nw, nchunk, _CHUNK)
    dst3 = dst.reshape(nw, nchunk, _CHUNK)
    w3 = w.reshape(nw, nchunk, _CHUNK)

    x2 = x.reshape(n_nodes, t_steps * d_in)
    wihT = W_ih.T
    whhT = W_hh.T
    b2 = (b_ih + b_hh).reshape(1, 4 * hid)
    gcnwT = gcn_W.T
    gcnb2 = gcn_b.reshape(1, hid)
    wg = out_W[:, :hid].T
    wh = out_W[:, hid:].T
    ob2 = out_b.reshape(1, 1)

    h, xw = _lstm_tc(x2, wihT, whhT, b2, gcnwT,
                     n_nodes, t_steps, d_in, hid, bn=400)
    degp = _sc_deg(dst3, w3, n_nodes)
    dis, y = _disy_tc(degp, xw, n_nodes, hid, bnb=2000)
    accp = _sc_msg(src3, dst3, w3, y, n_nodes, hid)
    out = _head_tc(accp, dis, xw, h, gcnb2, wg, wh, ob2,
                   n_nodes, hid, bnb=2000)
    return (out, 0)


# speedup vs baseline: 4.1855x; 1.0205x over previous
import functools

import jax
import jax.numpy as jnp
from jax import lax
from jax.experimental import pallas as pl
from jax.experimental.pallas import tpu as pltpu
from jax.experimental.pallas import tpu_sc as plsc

_NC = 2
_NS = 16
_CHUNK = 128


def _lstm_tc(x2, wihT, whhT, b2, gcnwT, n_nodes, t_steps, d_in, hid, bn):
    nb = n_nodes // bn

    def body(x_ref, wih_ref, whh_ref, b_ref, gcnw_ref, h_ref, xw_ref):
        h = jnp.zeros((bn, hid), jnp.float32)
        c = jnp.zeros((bn, hid), jnp.float32)
        wih = wih_ref[...]
        whh = whh_ref[...]
        b = b_ref[...]
        for t in range(t_steps):
            xt = x_ref[:, t, :]
            gates = (jnp.dot(xt, wih, preferred_element_type=jnp.float32)
                     + jnp.dot(h, whh, preferred_element_type=jnp.float32)
                     + b)
            i = jax.nn.sigmoid(gates[:, :hid])
            f = jax.nn.sigmoid(gates[:, hid:2 * hid])
            g = jnp.tanh(gates[:, 2 * hid:3 * hid])
            o = jax.nn.sigmoid(gates[:, 3 * hid:])
            c = f * c + i * g
            h = o * jnp.tanh(c)
        h_ref[...] = h
        xw_ref[...] = jnp.dot(h, gcnw_ref[...], preferred_element_type=jnp.float32)

    return pl.pallas_call(
        body,
        grid=(nb,),
        in_specs=[
            pl.BlockSpec((bn, t_steps, d_in), lambda i: (i, 0, 0)),
            pl.BlockSpec((d_in, 4 * hid), lambda i: (0, 0)),
            pl.BlockSpec((hid, 4 * hid), lambda i: (0, 0)),
            pl.BlockSpec((1, 4 * hid), lambda i: (0, 0)),
            pl.BlockSpec((hid, hid), lambda i: (0, 0)),
        ],
        out_specs=[
            pl.BlockSpec((bn, hid), lambda i: (i, 0)),
            pl.BlockSpec((bn, hid), lambda i: (i, 0)),
        ],
        out_shape=[
            jax.ShapeDtypeStruct((n_nodes, hid), jnp.float32),
            jax.ShapeDtypeStruct((n_nodes, hid), jnp.float32),
        ],
        compiler_params=pltpu.CompilerParams(
            dimension_semantics=("arbitrary",)),
    )(x2, wihT, whhT, b2, gcnwT)


def _sc_deg(dst3, w3, n_nodes):
    nchunk = dst3.shape[1]
    base = (n_nodes // (8 * _NS)) * 8
    last = n_nodes - (_NS - 1) * base
    mesh = plsc.VectorSubcoreMesh(
        core_axis_name="c", subcore_axis_name="s",
        num_cores=_NC, num_subcores=_NS)

    @functools.partial(
        pl.kernel,
        out_type=pltpu.HBM((_NC, n_nodes, 128), jnp.float32),
        mesh=mesh,
        scratch_types=[
            pltpu.VMEM((nchunk, _CHUNK), jnp.int32),
            pltpu.VMEM((nchunk, _CHUNK), jnp.float32),
            pltpu.VMEM((_CHUNK, 128), jnp.float32),
            pltpu.VMEM((16, 128), jnp.float32),
            pltpu.VMEM_SHARED((n_nodes, 128), jnp.float32),
        ],
    )
    def k(dst_hbm, w_hbm, out_hbm, dstv, wv, rows, zbuf, deg_s):
        cid = lax.axis_index("c")
        sid = lax.axis_index("s")
        wid = cid * _NS + sid
        pltpu.sync_copy(dst_hbm.at[wid], dstv)
        pltpu.sync_copy(w_hbm.at[wid], wv)

        zv = jnp.zeros((16,), jnp.float32)
        for r in range(16):
            for q in range(8):
                zbuf[r, pl.ds(q * 16, 16)] = zv
        def zrows(e, carry):
            for q in range(1, 8):
                rows[e, pl.ds(q * 16, 16)] = zv
            return carry
        lax.fori_loop(0, _CHUNK, zrows, 0)
        nz = jnp.where(sid == _NS - 1, last // 16, base // 16)

        def zero_chunk(i, carry):
            pltpu.sync_copy(zbuf, deg_s.at[pl.ds(sid * base + i * 16, 16)])
            return carry
        lax.fori_loop(0, nz, zero_chunk, 0)
        plsc.subcore_barrier()

        def chunk_body(j, carry):
            def fill(g, c2):
                wvec = wv[j, pl.ds(g * 16, 16)]
                for l in range(16):
                    rows[g * 16 + l, pl.ds(0, 16)] = jnp.full(
                        (16,), wvec[l], jnp.float32)
                return c2
            lax.fori_loop(0, _CHUNK // 16, fill, 0)
            pltpu.sync_copy(rows, deg_s.at[dstv.at[j]], add=True)
            return carry
        lax.fori_loop(0, nchunk, chunk_body, 0)
        plsc.subcore_barrier()

        @pl.when(sid < _NS - 1)
        def _():
            pltpu.sync_copy(deg_s.at[pl.ds(sid * base, base)],
                            out_hbm.at[cid, pl.ds(sid * base, base)])

        @pl.when(sid == _NS - 1)
        def _():
            pltpu.sync_copy(deg_s.at[pl.ds((_NS - 1) * base, last)],
                            out_hbm.at[cid, pl.ds((_NS - 1) * base, last)])

    return k(dst3, w3)


def _disy_tc(degp, xw, n_nodes, hid, bnb):
    nb = n_nodes // bnb

    def body(dp_ref, xw_ref, dis_ref, y_ref):
        dp = dp_ref[:, :, :16]
        deg = jnp.sum(dp, axis=(0, 2)) * (1.0 / 16.0) + 1.0
        dis = jnp.where(deg > 0, lax.rsqrt(deg), 0.0)
        dis_ref[...] = dis[:, None]
        y_ref[:, :hid] = xw_ref[...] * dis[:, None]
        y_ref[:, hid:] = jnp.zeros((y_ref.shape[0], 128 - hid), jnp.float32)

    return pl.pallas_call(
        body,
        grid=(nb,),
        in_specs=[
            pl.BlockSpec((_NC, bnb, 128), lambda i: (0, i, 0)),
            pl.BlockSpec((bnb, hid), lambda i: (i, 0)),
        ],
        out_specs=[
            pl.BlockSpec((bnb, 1), lambda i: (i, 0)),
            pl.BlockSpec((bnb, 128), lambda i: (i, 0)),
        ],
        out_shape=[
            jax.ShapeDtypeStruct((n_nodes, 1), jnp.float32),
            jax.ShapeDtypeStruct((n_nodes, 128), jnp.float32),
        ],
    )(degp, xw)


def _sc_msg(src3, dst3, w3, y, n_nodes, hid):
    nchunk = src3.shape[1]
    base = (n_nodes // (8 * _NS)) * 8
    last = n_nodes - (_NS - 1) * base
    mesh = plsc.VectorSubcoreMesh(
        core_axis_name="c", subcore_axis_name="s",
        num_cores=_NC, num_subcores=_NS)

    @functools.partial(
        pl.kernel,
        out_type=pltpu.HBM((_NC, n_nodes, 128), jnp.float32),
        mesh=mesh,
        scratch_types=[
            pltpu.VMEM((nchunk, _CHUNK), jnp.int32),
            pltpu.VMEM((nchunk, _CHUNK), jnp.int32),
            pltpu.VMEM((nchunk, _CHUNK), jnp.float32),
            pltpu.VMEM((_CHUNK, 128), jnp.float32),
            pltpu.VMEM((16, 128), jnp.float32),
            pltpu.VMEM_SHARED((n_nodes, 128), jnp.float32),
            pltpu.SemaphoreType.DMA,
        ],
    )
    def k(src_hbm, dst_hbm, w_hbm, y_hbm, out_hbm,
          srcv, dstv, wv, rows, zbuf, acc_s, sem):
        cid = lax.axis_index("c")
        sid = lax.axis_index("s")
        wid = cid * _NS + sid
        pltpu.sync_copy(src_hbm.at[wid], srcv)
        pltpu.sync_copy(dst_hbm.at[wid], dstv)
        pltpu.sync_copy(w_hbm.at[wid], wv)

        zv = jnp.zeros((16,), jnp.float32)
        for r in range(16):
            for q in range(8):
                zbuf[r, pl.ds(q * 16, 16)] = zv
        nz = jnp.where(sid == _NS - 1, last // 16, base // 16)

        def zero_chunk(i, carry):
            pltpu.sync_copy(zbuf, acc_s.at[pl.ds(sid * base + i * 16, 16)])
            return carry
        lax.fori_loop(0, nz, zero_chunk, 0)
        plsc.subcore_barrier()

        def chunk_body(j, carry):
            pltpu.async_copy(y_hbm.at[srcv.at[j]], rows, sem).wait()

            def scale(g, c2):
                wvec = wv[j, pl.ds(g * 16, 16)]
                for l in range(16):
                    e = g * 16 + l
                    w = wvec[l]
                    rows[e, pl.ds(0, 16)] = rows[e, pl.ds(0, 16)] * w
                    rows[e, pl.ds(16, 16)] = rows[e, pl.ds(16, 16)] * w
                return c2
            lax.fori_loop(0, _CHUNK // 16, scale, 0)
            pltpu.sync_copy(rows, acc_s.at[dstv.at[j]], add=True)
            return carry
        lax.fori_loop(0, nchunk, chunk_body, 0)
        plsc.subcore_barrier()

        @pl.when(sid < _NS - 1)
        def _():
            pltpu.sync_copy(acc_s.at[pl.ds(sid * base, base)],
                            out_hbm.at[cid, pl.ds(sid * base, base)])

        @pl.when(sid == _NS - 1)
        def _():
            pltpu.sync_copy(acc_s.at[pl.ds((_NS - 1) * base, last)],
                            out_hbm.at[cid, pl.ds((_NS - 1) * base, last)])

    return k(src3, dst3, w3, y)


def _head_tc(accp, dis, xw, h, gcnb2, wg, wh, ob2, n_nodes, hid, bnb):
    nb = n_nodes // bnb

    def body(a_ref, dis_ref, xw_ref, h_ref, gcnb_ref, wg_ref, wh_ref, ob_ref,
             o_ref):
        dis_c = dis_ref[...]
        acc = a_ref[0, :, :hid] + a_ref[1, :, :hid]
        gcn = (acc * dis_c
               + xw_ref[...] * (dis_c * dis_c) + gcnb_ref[...])
        zg = jnp.where(gcn > 0, gcn, jnp.exp(jnp.minimum(gcn, 0.0)) - 1.0)
        hh = h_ref[...]
        zh = jnp.where(hh > 0, hh, jnp.exp(jnp.minimum(hh, 0.0)) - 1.0)
        s = (jnp.dot(zg, wg_ref[...], preferred_element_type=jnp.float32)
             + jnp.dot(zh, wh_ref[...], preferred_element_type=jnp.float32)
             + ob_ref[...])
        o_ref[...] = jax.nn.sigmoid(s)

    return pl.pallas_call(
        body,
        grid=(nb,),
        in_specs=[
            pl.BlockSpec((_NC, bnb, 128), lambda i: (0, i, 0)),
            pl.BlockSpec((bnb, 1), lambda i: (i, 0)),
            pl.BlockSpec((bnb, hid), lambda i: (i, 0)),
            pl.BlockSpec((bnb, hid), lambda i: (i, 0)),
            pl.BlockSpec((1, hid), lambda i: (0, 0)),
            pl.BlockSpec((hid, 1), lambda i: (0, 0)),
            pl.BlockSpec((hid, 1), lambda i: (0, 0)),
            pl.BlockSpec((1, 1), lambda i: (0, 0)),
        ],
        out_specs=pl.BlockSpec((bnb, 1), lambda i: (i, 0)),
        out_shape=jax.ShapeDtypeStruct((n_nodes, 1), jnp.float32),
    )(accp, dis, xw, h, gcnb2, wg, wh, ob2)


def kernel(x, edge_index, edge_attr, W_ih, W_hh, b_ih, b_hh,
           gcn_W, gcn_b, out_W, out_b):
    n_nodes, t_steps, d_in = x.shape
    hid = W_hh.shape[1]
    e_edges = edge_attr.shape[0]
    nw = _NC * _NS

    grp = nw * _CHUNK
    epad = ((e_edges + grp - 1) // grp) * grp
    pad = epad - e_edges
    src = edge_index[0].astype(jnp.int32)
    dst = edge_index[1].astype(jnp.int32)
    w = edge_attr
    if pad:
        src = jnp.concatenate([src, jnp.zeros((pad,), jnp.int32)])
        dst = jnp.concatenate([dst, jnp.zeros((pad,), jnp.int32)])
        w = jnp.concatenate([w, jnp.zeros((pad,), w.dtype)])
    nchunk = epad // grp
    src3 = src.reshape(nw, nchunk, _CHUNK)
    dst3 = dst.reshape(nw, nchunk, _CHUNK)
    w3 = w.reshape(nw, nchunk, _CHUNK)

    wihT = W_ih.T
    whhT = W_hh.T
    b2 = (b_ih + b_hh).reshape(1, 4 * hid)
    gcnwT = gcn_W.T
    gcnb2 = gcn_b.reshape(1, hid)
    wg = out_W[:, :hid].T
    wh = out_W[:, hid:].T
    ob2 = out_b.reshape(1, 1)

    h, xw = _lstm_tc(x, wihT, whhT, b2, gcnwT,
                     n_nodes, t_steps, d_in, hid, bn=400)
    degp = _sc_deg(dst3, w3, n_nodes)
    dis, y = _disy_tc(degp, xw, n_nodes, hid, bnb=2000)
    accp = _sc_msg(src3, dst3, w3, y, n_nodes, hid)
    out = _head_tc(accp, dis, xw, h, gcnb2, wg, wh, ob2,
                   n_nodes, hid, bnb=2000)
    return (out, 0)


# speedup vs baseline: 4.3999x; 1.0512x over previous
import functools

import jax
import jax.numpy as jnp
from jax import lax
from jax.experimental import pallas as pl
from jax.experimental.pallas import tpu as pltpu
from jax.experimental.pallas import tpu_sc as plsc

_NC = 2
_NS = 16
_CHUNK = 128


def _lstm_tc(x2, wihT, whhT, b2, gcnwT, n_nodes, t_steps, d_in, hid, bn):
    nb = n_nodes // bn

    def body(x_ref, wih_ref, whh_ref, b_ref, gcnw_ref, h_ref, xw_ref):
        h = jnp.zeros((bn, hid), jnp.float32)
        c = jnp.zeros((bn, hid), jnp.float32)
        wih = wih_ref[...]
        whh = whh_ref[...]
        b = b_ref[...]
        for t in range(t_steps):
            xt = x_ref[:, t, :]
            gates = (jnp.dot(xt, wih, preferred_element_type=jnp.float32)
                     + jnp.dot(h, whh, preferred_element_type=jnp.float32)
                     + b)
            i = jax.nn.sigmoid(gates[:, :hid])
            f = jax.nn.sigmoid(gates[:, hid:2 * hid])
            g = jnp.tanh(gates[:, 2 * hid:3 * hid])
            o = jax.nn.sigmoid(gates[:, 3 * hid:])
            c = f * c + i * g
            h = o * jnp.tanh(c)
        h_ref[...] = h
        xw_ref[...] = jnp.dot(h, gcnw_ref[...], preferred_element_type=jnp.float32)

    return pl.pallas_call(
        body,
        grid=(nb,),
        in_specs=[
            pl.BlockSpec((bn, t_steps, d_in), lambda i: (i, 0, 0)),
            pl.BlockSpec((d_in, 4 * hid), lambda i: (0, 0)),
            pl.BlockSpec((hid, 4 * hid), lambda i: (0, 0)),
            pl.BlockSpec((1, 4 * hid), lambda i: (0, 0)),
            pl.BlockSpec((hid, hid), lambda i: (0, 0)),
        ],
        out_specs=[
            pl.BlockSpec((bn, hid), lambda i: (i, 0)),
            pl.BlockSpec((bn, hid), lambda i: (i, 0)),
        ],
        out_shape=[
            jax.ShapeDtypeStruct((n_nodes, hid), jnp.float32),
            jax.ShapeDtypeStruct((n_nodes, hid), jnp.float32),
        ],
        compiler_params=pltpu.CompilerParams(
            dimension_semantics=("arbitrary",)),
    )(x2, wihT, whhT, b2, gcnwT)


def _sc_deg(dst3, w3, n_nodes):
    nchunk = dst3.shape[1]
    base = (n_nodes // (8 * _NS)) * 8
    last = n_nodes - (_NS - 1) * base
    mesh = plsc.VectorSubcoreMesh(
        core_axis_name="c", subcore_axis_name="s",
        num_cores=_NC, num_subcores=_NS)

    @functools.partial(
        pl.kernel,
        out_type=pltpu.HBM((_NC, n_nodes, 128), jnp.float32),
        mesh=mesh,
        scratch_types=[
            pltpu.VMEM((nchunk, _CHUNK), jnp.int32),
            pltpu.VMEM((nchunk, _CHUNK), jnp.float32),
            pltpu.VMEM((_CHUNK, 128), jnp.float32),
            pltpu.VMEM((_CHUNK, 128), jnp.float32),
            pltpu.VMEM((16, 128), jnp.float32),
            pltpu.VMEM_SHARED((n_nodes, 128), jnp.float32),
            pltpu.SemaphoreType.DMA,
            pltpu.SemaphoreType.DMA,
        ],
    )
    def k(dst_hbm, w_hbm, out_hbm, dstv, wv, rows0, rows1, zbuf, deg_s,
          ssem0, ssem1):
        cid = lax.axis_index("c")
        sid = lax.axis_index("s")
        wid = cid * _NS + sid
        pltpu.sync_copy(dst_hbm.at[wid], dstv)
        pltpu.sync_copy(w_hbm.at[wid], wv)

        zv = jnp.zeros((16,), jnp.float32)
        for r in range(16):
            for q in range(8):
                zbuf[r, pl.ds(q * 16, 16)] = zv
        def zrows(e, carry):
            for q in range(1, 8):
                rows0[e, pl.ds(q * 16, 16)] = zv
                rows1[e, pl.ds(q * 16, 16)] = zv
            return carry
        lax.fori_loop(0, _CHUNK, zrows, 0)
        nz = jnp.where(sid == _NS - 1, last // 16, base // 16)

        def zero_chunk(i, carry):
            pltpu.sync_copy(zbuf, deg_s.at[pl.ds(sid * base + i * 16, 16)])
            return carry
        lax.fori_loop(0, nz, zero_chunk, 0)
        plsc.subcore_barrier()

        bufs = ((rows0, ssem0), (rows1, ssem1))

        def fill(j, rows):
            def fill_g(g, c2):
                wvec = wv[j, pl.ds(g * 16, 16)]
                for l in range(16):
                    rows[g * 16 + l, pl.ds(0, 16)] = jnp.full(
                        (16,), wvec[l], jnp.float32)
                return c2
            lax.fori_loop(0, _CHUNK // 16, fill_g, 0)

        fill(0, rows0)

        def chunk_body(j, carry):
            for b in range(2):
                rows, ssem = bufs[b]
                nrows, nssem = bufs[1 - b]

                @pl.when(j % 2 == b)
                def _():
                    @pl.when(j >= 1)
                    def _():
                        pltpu.make_async_copy(
                            nrows, deg_s.at[dstv.at[j - 1]], nssem).wait()
                    pltpu.async_copy(rows, deg_s.at[dstv.at[j]], ssem,
                                     add=True)

                    @pl.when(j + 1 < nchunk)
                    def _():
                        fill(j + 1, nrows)
            return carry
        lax.fori_loop(0, nchunk, chunk_body, 0)
        last_b = (nchunk - 1) % 2
        pltpu.make_async_copy(bufs[last_b][0],
                              deg_s.at[dstv.at[nchunk - 1]],
                              bufs[last_b][1]).wait()
        plsc.subcore_barrier()

        @pl.when(sid < _NS - 1)
        def _():
            pltpu.sync_copy(deg_s.at[pl.ds(sid * base, base)],
                            out_hbm.at[cid, pl.ds(sid * base, base)])

        @pl.when(sid == _NS - 1)
        def _():
            pltpu.sync_copy(deg_s.at[pl.ds((_NS - 1) * base, last)],
                            out_hbm.at[cid, pl.ds((_NS - 1) * base, last)])

    return k(dst3, w3)


def _disy_tc(degp, xw, n_nodes, hid, bnb):
    nb = n_nodes // bnb

    def body(dp_ref, xw_ref, dis_ref, y_ref):
        dp = dp_ref[:, :, :16]
        deg = jnp.sum(dp, axis=(0, 2)) * (1.0 / 16.0) + 1.0
        dis = jnp.where(deg > 0, lax.rsqrt(deg), 0.0)
        dis_ref[...] = dis[:, None]
        y_ref[:, :hid] = xw_ref[...] * dis[:, None]
        y_ref[:, hid:] = jnp.zeros((y_ref.shape[0], 128 - hid), jnp.float32)

    return pl.pallas_call(
        body,
        grid=(nb,),
        in_specs=[
            pl.BlockSpec((_NC, bnb, 128), lambda i: (0, i, 0)),
            pl.BlockSpec((bnb, hid), lambda i: (i, 0)),
        ],
        out_specs=[
            pl.BlockSpec((bnb, 1), lambda i: (i, 0)),
            pl.BlockSpec((bnb, 128), lambda i: (i, 0)),
        ],
        out_shape=[
            jax.ShapeDtypeStruct((n_nodes, 1), jnp.float32),
            jax.ShapeDtypeStruct((n_nodes, 128), jnp.float32),
        ],
    )(degp, xw)


def _sc_msg(src3, dst3, w3, y, n_nodes, hid):
    nchunk = src3.shape[1]
    base = (n_nodes // (8 * _NS)) * 8
    last = n_nodes - (_NS - 1) * base
    mesh = plsc.VectorSubcoreMesh(
        core_axis_name="c", subcore_axis_name="s",
        num_cores=_NC, num_subcores=_NS)

    @functools.partial(
        pl.kernel,
        out_type=pltpu.HBM((_NC, n_nodes, 128), jnp.float32),
        mesh=mesh,
        scratch_types=[
            pltpu.VMEM((nchunk, _CHUNK), jnp.int32),
            pltpu.VMEM((nchunk, _CHUNK), jnp.int32),
            pltpu.VMEM((nchunk, _CHUNK), jnp.float32),
            pltpu.VMEM((_CHUNK, 128), jnp.float32),
            pltpu.VMEM((_CHUNK, 128), jnp.float32),
            pltpu.VMEM((16, 128), jnp.float32),
            pltpu.VMEM_SHARED((n_nodes, 128), jnp.float32),
            pltpu.SemaphoreType.DMA,
            pltpu.SemaphoreType.DMA,
            pltpu.SemaphoreType.DMA,
            pltpu.SemaphoreType.DMA,
        ],
    )
    def k(src_hbm, dst_hbm, w_hbm, y_hbm, out_hbm,
          srcv, dstv, wv, rows0, rows1, zbuf, acc_s,
          gsem0, gsem1, ssem0, ssem1):
        cid = lax.axis_index("c")
        sid = lax.axis_index("s")
        wid = cid * _NS + sid
        pltpu.sync_copy(src_hbm.at[wid], srcv)
        pltpu.sync_copy(dst_hbm.at[wid], dstv)
        pltpu.sync_copy(w_hbm.at[wid], wv)

        zv = jnp.zeros((16,), jnp.float32)
        for r in range(16):
            for q in range(8):
                zbuf[r, pl.ds(q * 16, 16)] = zv
        nz = jnp.where(sid == _NS - 1, last // 16, base // 16)

        def zero_chunk(i, carry):
            pltpu.sync_copy(zbuf, acc_s.at[pl.ds(sid * base + i * 16, 16)])
            return carry
        lax.fori_loop(0, nz, zero_chunk, 0)
        plsc.subcore_barrier()

        bufs = ((rows0, gsem0, ssem0), (rows1, gsem1, ssem1))

        def gather_start(j, rows, gsem):
            pltpu.async_copy(y_hbm.at[srcv.at[j]], rows, gsem)

        def scale_and_scatter(j, rows, gsem, ssem):
            pltpu.make_async_copy(y_hbm.at[srcv.at[j]], rows, gsem).wait()

            def scale(g, c2):
                wvec = wv[j, pl.ds(g * 16, 16)]
                for l in range(16):
                    e = g * 16 + l
                    w = wvec[l]
                    rows[e, pl.ds(0, 16)] = rows[e, pl.ds(0, 16)] * w
                    rows[e, pl.ds(16, 16)] = rows[e, pl.ds(16, 16)] * w
                return c2
            lax.fori_loop(0, _CHUNK // 16, scale, 0)
            pltpu.async_copy(rows, acc_s.at[dstv.at[j]], ssem, add=True)

        gather_start(0, rows0, gsem0)

        def chunk_body(j, carry):
            for b in range(2):
                rows, gsem, ssem = bufs[b]
                nrows, ngsem, nssem = bufs[1 - b]

                @pl.when(j % 2 == b)
                def _():
                    @pl.when(j >= 1)
                    def _():
                        pltpu.make_async_copy(
                            nrows, acc_s.at[dstv.at[j - 1]], nssem).wait()

                    @pl.when(j + 1 < nchunk)
                    def _():
                        gather_start(j + 1, nrows, ngsem)
                    scale_and_scatter(j, rows, gsem, ssem)
            return carry
        lax.fori_loop(0, nchunk, chunk_body, 0)
        last_b = (nchunk - 1) % 2
        pltpu.make_async_copy(bufs[last_b][0],
                              acc_s.at[dstv.at[nchunk - 1]],
                              bufs[last_b][2]).wait()
        plsc.subcore_barrier()

        @pl.when(sid < _NS - 1)
        def _():
            pltpu.sync_copy(acc_s.at[pl.ds(sid * base, base)],
                            out_hbm.at[cid, pl.ds(sid * base, base)])

        @pl.when(sid == _NS - 1)
        def _():
            pltpu.sync_copy(acc_s.at[pl.ds((_NS - 1) * base, last)],
                            out_hbm.at[cid, pl.ds((_NS - 1) * base, last)])

    return k(src3, dst3, w3, y)


def _head_tc(accp, dis, xw, h, gcnb2, wg, wh, ob2, n_nodes, hid, bnb):
    nb = n_nodes // bnb

    def body(a_ref, dis_ref, xw_ref, h_ref, gcnb_ref, wg_ref, wh_ref, ob_ref,
             o_ref):
        dis_c = dis_ref[...]
        acc = a_ref[0, :, :hid] + a_ref[1, :, :hid]
        gcn = (acc * dis_c
               + xw_ref[...] * (dis_c * dis_c) + gcnb_ref[...])
        zg = jnp.where(gcn > 0, gcn, jnp.exp(jnp.minimum(gcn, 0.0)) - 1.0)
        hh = h_ref[...]
        zh = jnp.where(hh > 0, hh, jnp.exp(jnp.minimum(hh, 0.0)) - 1.0)
        s = (jnp.dot(zg, wg_ref[...], preferred_element_type=jnp.float32)
             + jnp.dot(zh, wh_ref[...], preferred_element_type=jnp.float32)
             + ob_ref[...])
        o_ref[...] = jax.nn.sigmoid(s)

    return pl.pallas_call(
        body,
        grid=(nb,),
        in_specs=[
            pl.BlockSpec((_NC, bnb, 128), lambda i: (0, i, 0)),
            pl.BlockSpec((bnb, 1), lambda i: (i, 0)),
            pl.BlockSpec((bnb, hid), lambda i: (i, 0)),
            pl.BlockSpec((bnb, hid), lambda i: (i, 0)),
            pl.BlockSpec((1, hid), lambda i: (0, 0)),
            pl.BlockSpec((hid, 1), lambda i: (0, 0)),
            pl.BlockSpec((hid, 1), lambda i: (0, 0)),
            pl.BlockSpec((1, 1), lambda i: (0, 0)),
        ],
        out_specs=pl.BlockSpec((bnb, 1), lambda i: (i, 0)),
        out_shape=jax.ShapeDtypeStruct((n_nodes, 1), jnp.float32),
    )(accp, dis, xw, h, gcnb2, wg, wh, ob2)


def kernel(x, edge_index, edge_attr, W_ih, W_hh, b_ih, b_hh,
           gcn_W, gcn_b, out_W, out_b):
    n_nodes, t_steps, d_in = x.shape
    hid = W_hh.shape[1]
    e_edges = edge_attr.shape[0]
    nw = _NC * _NS

    grp = nw * _CHUNK
    epad = ((e_edges + grp - 1) // grp) * grp
    pad = epad - e_edges
    src = edge_index[0].astype(jnp.int32)
    dst = edge_index[1].astype(jnp.int32)
    w = edge_attr
    if pad:
        src = jnp.concatenate([src, jnp.zeros((pad,), jnp.int32)])
        dst = jnp.concatenate([dst, jnp.zeros((pad,), jnp.int32)])
        w = jnp.concatenate([w, jnp.zeros((pad,), w.dtype)])
    nchunk = epad // grp
    src3 = src.reshape(nw, nchunk, _CHUNK)
    dst3 = dst.reshape(nw, nchunk, _CHUNK)
    w3 = w.reshape(nw, nchunk, _CHUNK)

    wihT = W_ih.T
    whhT = W_hh.T
    b2 = (b_ih + b_hh).reshape(1, 4 * hid)
    gcnwT = gcn_W.T
    gcnb2 = gcn_b.reshape(1, hid)
    wg = out_W[:, :hid].T
    wh = out_W[:, hid:].T
    ob2 = out_b.reshape(1, 1)

    h, xw = _lstm_tc(x, wihT, whhT, b2, gcnwT,
                     n_nodes, t_steps, d_in, hid, bn=400)
    degp = _sc_deg(dst3, w3, n_nodes)
    dis, y = _disy_tc(degp, xw, n_nodes, hid, bnb=2000)
    accp = _sc_msg(src3, dst3, w3, y, n_nodes, hid)
    out = _head_tc(accp, dis, xw, h, gcnb2, wg, wh, ob2,
                   n_nodes, hid, bnb=2000)
    return (out, 0)


# speedup vs baseline: 7.1364x; 1.6219x over previous
import functools

import jax
import jax.numpy as jnp
from jax import lax
from jax.experimental import pallas as pl
from jax.experimental.pallas import tpu as pltpu
from jax.experimental.pallas import tpu_sc as plsc

_NC = 2
_NS = 16
_CHUNK = 128


def _lstm_tc(x2, wihT, whhT, b2, gcnwT, n_nodes, t_steps, d_in, hid, bn):
    nb = n_nodes // bn

    def body(x_ref, wih_ref, whh_ref, b_ref, gcnw_ref, h_ref, xw_ref):
        h = jnp.zeros((bn, hid), jnp.float32)
        c = jnp.zeros((bn, hid), jnp.float32)
        wih = wih_ref[...]
        whh = whh_ref[...]
        b = b_ref[...]
        for t in range(t_steps):
            xt = x_ref[t]
            gates = (jnp.dot(xt, wih, preferred_element_type=jnp.float32)
                     + jnp.dot(h, whh, preferred_element_type=jnp.float32)
                     + b)
            i = jax.nn.sigmoid(gates[:, :hid])
            f = jax.nn.sigmoid(gates[:, hid:2 * hid])
            g = jnp.tanh(gates[:, 2 * hid:3 * hid])
            o = jax.nn.sigmoid(gates[:, 3 * hid:])
            c = f * c + i * g
            h = o * jnp.tanh(c)
        h_ref[...] = h
        xw_ref[...] = jnp.dot(h, gcnw_ref[...], preferred_element_type=jnp.float32)

    return pl.pallas_call(
        body,
        grid=(nb,),
        in_specs=[
            pl.BlockSpec((t_steps, bn, d_in), lambda i: (0, i, 0)),
            pl.BlockSpec((d_in, 4 * hid), lambda i: (0, 0)),
            pl.BlockSpec((hid, 4 * hid), lambda i: (0, 0)),
            pl.BlockSpec((1, 4 * hid), lambda i: (0, 0)),
            pl.BlockSpec((hid, hid), lambda i: (0, 0)),
        ],
        out_specs=[
            pl.BlockSpec((bn, hid), lambda i: (i, 0)),
            pl.BlockSpec((bn, hid), lambda i: (i, 0)),
        ],
        out_shape=[
            jax.ShapeDtypeStruct((n_nodes, hid), jnp.float32),
            jax.ShapeDtypeStruct((n_nodes, hid), jnp.float32),
        ],
        compiler_params=pltpu.CompilerParams(
            dimension_semantics=("arbitrary",)),
    )(x2, wihT, whhT, b2, gcnwT)


def _sc_deg(dst3, w3, n_nodes):
    nchunk = dst3.shape[1]
    base = (n_nodes // (8 * _NS)) * 8
    last = n_nodes - (_NS - 1) * base
    mesh = plsc.VectorSubcoreMesh(
        core_axis_name="c", subcore_axis_name="s",
        num_cores=_NC, num_subcores=_NS)

    @functools.partial(
        pl.kernel,
        out_type=pltpu.HBM((_NC, n_nodes, 128), jnp.float32),
        mesh=mesh,
        scratch_types=[
            pltpu.VMEM((nchunk, _CHUNK), jnp.int32),
            pltpu.VMEM((nchunk, _CHUNK), jnp.float32),
            pltpu.VMEM((_CHUNK, 128), jnp.float32),
            pltpu.VMEM((_CHUNK, 128), jnp.float32),
            pltpu.VMEM((16, 128), jnp.float32),
            pltpu.VMEM_SHARED((n_nodes, 128), jnp.float32),
            pltpu.SemaphoreType.DMA,
            pltpu.SemaphoreType.DMA,
        ],
    )
    def k(dst_hbm, w_hbm, out_hbm, dstv, wv, rows0, rows1, zbuf, deg_s,
          ssem0, ssem1):
        cid = lax.axis_index("c")
        sid = lax.axis_index("s")
        wid = cid * _NS + sid
        pltpu.sync_copy(dst_hbm.at[wid], dstv)
        pltpu.sync_copy(w_hbm.at[wid], wv)

        zv = jnp.zeros((16,), jnp.float32)
        for r in range(16):
            for q in range(8):
                zbuf[r, pl.ds(q * 16, 16)] = zv
        def zrows(e, carry):
            for q in range(1, 8):
                rows0[e, pl.ds(q * 16, 16)] = zv
                rows1[e, pl.ds(q * 16, 16)] = zv
            return carry
        lax.fori_loop(0, _CHUNK, zrows, 0)
        nz = jnp.where(sid == _NS - 1, last // 16, base // 16)

        def zero_chunk(i, carry):
            pltpu.sync_copy(zbuf, deg_s.at[pl.ds(sid * base + i * 16, 16)])
            return carry
        lax.fori_loop(0, nz, zero_chunk, 0)
        plsc.subcore_barrier()

        bufs = ((rows0, ssem0), (rows1, ssem1))

        def fill(j, rows):
            def fill_g(g, c2):
                wvec = wv[j, pl.ds(g * 16, 16)]
                for l in range(16):
                    rows[g * 16 + l, pl.ds(0, 16)] = jnp.full(
                        (16,), wvec[l], jnp.float32)
                return c2
            lax.fori_loop(0, _CHUNK // 16, fill_g, 0)

        fill(0, rows0)

        def chunk_body(j, carry):
            for b in range(2):
                rows, ssem = bufs[b]
                nrows, nssem = bufs[1 - b]

                @pl.when(j % 2 == b)
                def _():
                    @pl.when(j >= 1)
                    def _():
                        pltpu.make_async_copy(
                            nrows, deg_s.at[dstv.at[j - 1]], nssem).wait()
                    pltpu.async_copy(rows, deg_s.at[dstv.at[j]], ssem,
                                     add=True)

                    @pl.when(j + 1 < nchunk)
                    def _():
                        fill(j + 1, nrows)
            return carry
        lax.fori_loop(0, nchunk, chunk_body, 0)
        last_b = (nchunk - 1) % 2
        pltpu.make_async_copy(bufs[last_b][0],
                              deg_s.at[dstv.at[nchunk - 1]],
                              bufs[last_b][1]).wait()
        plsc.subcore_barrier()

        @pl.when(sid < _NS - 1)
        def _():
            pltpu.sync_copy(deg_s.at[pl.ds(sid * base, base)],
                            out_hbm.at[cid, pl.ds(sid * base, base)])

        @pl.when(sid == _NS - 1)
        def _():
            pltpu.sync_copy(deg_s.at[pl.ds((_NS - 1) * base, last)],
                            out_hbm.at[cid, pl.ds((_NS - 1) * base, last)])

    return k(dst3, w3)


def _disy_tc(degp, xw, n_nodes, hid, bnb):
    nb = n_nodes // bnb

    def body(dp_ref, xw_ref, dis_ref, y_ref):
        dp = dp_ref[:, :, :16]
        deg = jnp.sum(dp, axis=(0, 2)) * (1.0 / 16.0) + 1.0
        dis = jnp.where(deg > 0, lax.rsqrt(deg), 0.0)
        dis_ref[...] = dis[:, None]
        y_ref[:, :hid] = xw_ref[...] * dis[:, None]
        y_ref[:, hid:] = jnp.zeros((y_ref.shape[0], 128 - hid), jnp.float32)

    return pl.pallas_call(
        body,
        grid=(nb,),
        in_specs=[
            pl.BlockSpec((_NC, bnb, 128), lambda i: (0, i, 0)),
            pl.BlockSpec((bnb, hid), lambda i: (i, 0)),
        ],
        out_specs=[
            pl.BlockSpec((bnb, 1), lambda i: (i, 0)),
            pl.BlockSpec((bnb, 128), lambda i: (i, 0)),
        ],
        out_shape=[
            jax.ShapeDtypeStruct((n_nodes, 1), jnp.float32),
            jax.ShapeDtypeStruct((n_nodes, 128), jnp.float32),
        ],
    )(degp, xw)


def _sc_msg(src3, dst3, w3, y, n_nodes, hid):
    nchunk = src3.shape[1]
    base = (n_nodes // (8 * _NS)) * 8
    last = n_nodes - (_NS - 1) * base
    mesh = plsc.VectorSubcoreMesh(
        core_axis_name="c", subcore_axis_name="s",
        num_cores=_NC, num_subcores=_NS)

    @functools.partial(
        pl.kernel,
        out_type=pltpu.HBM((_NC, n_nodes, 128), jnp.float32),
        mesh=mesh,
        scratch_types=[
            pltpu.VMEM((nchunk, _CHUNK), jnp.int32),
            pltpu.VMEM((nchunk, _CHUNK), jnp.int32),
            pltpu.VMEM((nchunk, _CHUNK), jnp.float32),
            pltpu.VMEM((_CHUNK, 128), jnp.float32),
            pltpu.VMEM((_CHUNK, 128), jnp.float32),
            pltpu.VMEM((16, 128), jnp.float32),
            pltpu.VMEM_SHARED((n_nodes, 128), jnp.float32),
            pltpu.SemaphoreType.DMA,
            pltpu.SemaphoreType.DMA,
            pltpu.SemaphoreType.DMA,
            pltpu.SemaphoreType.DMA,
        ],
    )
    def k(src_hbm, dst_hbm, w_hbm, y_hbm, out_hbm,
          srcv, dstv, wv, rows0, rows1, zbuf, acc_s,
          gsem0, gsem1, ssem0, ssem1):
        cid = lax.axis_index("c")
        sid = lax.axis_index("s")
        wid = cid * _NS + sid
        pltpu.sync_copy(src_hbm.at[wid], srcv)
        pltpu.sync_copy(dst_hbm.at[wid], dstv)
        pltpu.sync_copy(w_hbm.at[wid], wv)

        zv = jnp.zeros((16,), jnp.float32)
        for r in range(16):
            for q in range(8):
                zbuf[r, pl.ds(q * 16, 16)] = zv
        nz = jnp.where(sid == _NS - 1, last // 16, base // 16)

        def zero_chunk(i, carry):
            pltpu.sync_copy(zbuf, acc_s.at[pl.ds(sid * base + i * 16, 16)])
            return carry
        lax.fori_loop(0, nz, zero_chunk, 0)
        plsc.subcore_barrier()

        bufs = ((rows0, gsem0, ssem0), (rows1, gsem1, ssem1))

        def gather_start(j, rows, gsem):
            pltpu.async_copy(y_hbm.at[srcv.at[j]], rows, gsem)

        def scale_and_scatter(j, rows, gsem, ssem):
            pltpu.make_async_copy(y_hbm.at[srcv.at[j]], rows, gsem).wait()

            def scale(g, c2):
                wvec = wv[j, pl.ds(g * 16, 16)]
                for l in range(16):
                    e = g * 16 + l
                    w = wvec[l]
                    rows[e, pl.ds(0, 16)] = rows[e, pl.ds(0, 16)] * w
                    rows[e, pl.ds(16, 16)] = rows[e, pl.ds(16, 16)] * w
                return c2
            lax.fori_loop(0, _CHUNK // 16, scale, 0)
            pltpu.async_copy(rows, acc_s.at[dstv.at[j]], ssem, add=True)

        gather_start(0, rows0, gsem0)

        def chunk_body(j, carry):
            for b in range(2):
                rows, gsem, ssem = bufs[b]
                nrows, ngsem, nssem = bufs[1 - b]

                @pl.when(j % 2 == b)
                def _():
                    @pl.when(j >= 1)
                    def _():
                        pltpu.make_async_copy(
                            nrows, acc_s.at[dstv.at[j - 1]], nssem).wait()

                    @pl.when(j + 1 < nchunk)
                    def _():
                        gather_start(j + 1, nrows, ngsem)
                    scale_and_scatter(j, rows, gsem, ssem)
            return carry
        lax.fori_loop(0, nchunk, chunk_body, 0)
        last_b = (nchunk - 1) % 2
        pltpu.make_async_copy(bufs[last_b][0],
                              acc_s.at[dstv.at[nchunk - 1]],
                              bufs[last_b][2]).wait()
        plsc.subcore_barrier()

        @pl.when(sid < _NS - 1)
        def _():
            pltpu.sync_copy(acc_s.at[pl.ds(sid * base, base)],
                            out_hbm.at[cid, pl.ds(sid * base, base)])

        @pl.when(sid == _NS - 1)
        def _():
            pltpu.sync_copy(acc_s.at[pl.ds((_NS - 1) * base, last)],
                            out_hbm.at[cid, pl.ds((_NS - 1) * base, last)])

    return k(src3, dst3, w3, y)


def _head_tc(accp, dis, xw, h, gcnb2, wg, wh, ob2, n_nodes, hid, bnb):
    nb = n_nodes // bnb

    def body(a_ref, dis_ref, xw_ref, h_ref, gcnb_ref, wg_ref, wh_ref, ob_ref,
             o_ref):
        dis_c = dis_ref[...]
        acc = a_ref[0, :, :hid] + a_ref[1, :, :hid]
        gcn = (acc * dis_c
               + xw_ref[...] * (dis_c * dis_c) + gcnb_ref[...])
        zg = jnp.where(gcn > 0, gcn, jnp.exp(jnp.minimum(gcn, 0.0)) - 1.0)
        hh = h_ref[...]
        zh = jnp.where(hh > 0, hh, jnp.exp(jnp.minimum(hh, 0.0)) - 1.0)
        s = (jnp.dot(zg, wg_ref[...], preferred_element_type=jnp.float32)
             + jnp.dot(zh, wh_ref[...], preferred_element_type=jnp.float32)
             + ob_ref[...])
        o_ref[...] = jax.nn.sigmoid(s)

    return pl.pallas_call(
        body,
        grid=(nb,),
        in_specs=[
            pl.BlockSpec((_NC, bnb, 128), lambda i: (0, i, 0)),
            pl.BlockSpec((bnb, 1), lambda i: (i, 0)),
            pl.BlockSpec((bnb, hid), lambda i: (i, 0)),
            pl.BlockSpec((bnb, hid), lambda i: (i, 0)),
            pl.BlockSpec((1, hid), lambda i: (0, 0)),
            pl.BlockSpec((hid, 1), lambda i: (0, 0)),
            pl.BlockSpec((hid, 1), lambda i: (0, 0)),
            pl.BlockSpec((1, 1), lambda i: (0, 0)),
        ],
        out_specs=pl.BlockSpec((bnb, 1), lambda i: (i, 0)),
        out_shape=jax.ShapeDtypeStruct((n_nodes, 1), jnp.float32),
    )(accp, dis, xw, h, gcnb2, wg, wh, ob2)


def kernel(x, edge_index, edge_attr, W_ih, W_hh, b_ih, b_hh,
           gcn_W, gcn_b, out_W, out_b):
    n_nodes, t_steps, d_in = x.shape
    hid = W_hh.shape[1]
    e_edges = edge_attr.shape[0]
    nw = _NC * _NS

    grp = nw * _CHUNK
    epad = ((e_edges + grp - 1) // grp) * grp
    pad = epad - e_edges
    src = edge_index[0].astype(jnp.int32)
    dst = edge_index[1].astype(jnp.int32)
    w = edge_attr
    if pad:
        src = jnp.concatenate([src, jnp.zeros((pad,), jnp.int32)])
        dst = jnp.concatenate([dst, jnp.zeros((pad,), jnp.int32)])
        w = jnp.concatenate([w, jnp.zeros((pad,), w.dtype)])
    nchunk = epad // grp
    src3 = src.reshape(nw, nchunk, _CHUNK)
    dst3 = dst.reshape(nw, nchunk, _CHUNK)
    w3 = w.reshape(nw, nchunk, _CHUNK)

    wihT = W_ih.T
    whhT = W_hh.T
    b2 = (b_ih + b_hh).reshape(1, 4 * hid)
    gcnwT = gcn_W.T
    gcnb2 = gcn_b.reshape(1, hid)
    wg = out_W[:, :hid].T
    wh = out_W[:, hid:].T
    ob2 = out_b.reshape(1, 1)

    xt3 = jnp.swapaxes(x, 0, 1)
    h, xw = _lstm_tc(xt3, wihT, whhT, b2, gcnwT,
                     n_nodes, t_steps, d_in, hid, bn=400)
    degp = _sc_deg(dst3, w3, n_nodes)
    dis, y = _disy_tc(degp, xw, n_nodes, hid, bnb=2000)
    accp = _sc_msg(src3, dst3, w3, y, n_nodes, hid)
    out = _head_tc(accp, dis, xw, h, gcnb2, wg, wh, ob2,
                   n_nodes, hid, bnb=2000)
    return (out, 0)


# speedup vs baseline: 7.4339x; 1.0417x over previous
import functools

import jax
import jax.numpy as jnp
from jax import lax
from jax.experimental import pallas as pl
from jax.experimental.pallas import tpu as pltpu
from jax.experimental.pallas import tpu_sc as plsc

_NC = 2
_NS = 16
_CHUNK = 128


def _lstm_tc(x2, wihT, whhT, b2, gcnwT, n_nodes, t_steps, d_in, hid, bn):
    nb = n_nodes // bn

    def body(x_ref, wih_ref, whh_ref, b_ref, gcnw_ref, h_ref, xw_ref):
        hp = jnp.zeros((bn, 128), jnp.float32)
        c = jnp.zeros((bn, hid), jnp.float32)
        zpad = jnp.zeros((bn, 128 - hid), jnp.float32)
        wih = wih_ref[...]
        whh = whh_ref[...]
        b = b_ref[...]
        h = None
        for t in range(t_steps):
            xt = x_ref[t]
            gates = (jnp.dot(xt, wih, preferred_element_type=jnp.float32)
                     + jnp.dot(hp, whh, preferred_element_type=jnp.float32)
                     + b)
            i = jax.nn.sigmoid(gates[:, :hid])
            f = jax.nn.sigmoid(gates[:, hid:2 * hid])
            g = jnp.tanh(gates[:, 2 * hid:3 * hid])
            o = jax.nn.sigmoid(gates[:, 3 * hid:])
            c = f * c + i * g
            h = o * jnp.tanh(c)
            hp = jnp.concatenate([h, zpad], axis=1)
        h_ref[...] = h
        xw_ref[...] = jnp.dot(h, gcnw_ref[...], preferred_element_type=jnp.float32)

    return pl.pallas_call(
        body,
        grid=(nb,),
        in_specs=[
            pl.BlockSpec((t_steps, bn, d_in), lambda i: (0, i, 0)),
            pl.BlockSpec((d_in, 4 * hid), lambda i: (0, 0)),
            pl.BlockSpec((128, 4 * hid), lambda i: (0, 0)),
            pl.BlockSpec((1, 4 * hid), lambda i: (0, 0)),
            pl.BlockSpec((hid, hid), lambda i: (0, 0)),
        ],
        out_specs=[
            pl.BlockSpec((bn, hid), lambda i: (i, 0)),
            pl.BlockSpec((bn, hid), lambda i: (i, 0)),
        ],
        out_shape=[
            jax.ShapeDtypeStruct((n_nodes, hid), jnp.float32),
            jax.ShapeDtypeStruct((n_nodes, hid), jnp.float32),
        ],
        compiler_params=pltpu.CompilerParams(
            dimension_semantics=("arbitrary",)),
    )(x2, wihT, whhT, b2, gcnwT)


def _sc_deg(dst3, w3, n_nodes):
    nchunk = dst3.shape[1]
    base = (n_nodes // (8 * _NS)) * 8
    last = n_nodes - (_NS - 1) * base
    mesh = plsc.VectorSubcoreMesh(
        core_axis_name="c", subcore_axis_name="s",
        num_cores=_NC, num_subcores=_NS)

    @functools.partial(
        pl.kernel,
        out_type=pltpu.HBM((_NC, n_nodes, 128), jnp.float32),
        mesh=mesh,
        scratch_types=[
            pltpu.VMEM((nchunk, _CHUNK), jnp.int32),
            pltpu.VMEM((nchunk, _CHUNK), jnp.float32),
            pltpu.VMEM((_CHUNK, 128), jnp.float32),
            pltpu.VMEM((_CHUNK, 128), jnp.float32),
            pltpu.VMEM((16, 128), jnp.float32),
            pltpu.VMEM_SHARED((n_nodes, 128), jnp.float32),
            pltpu.SemaphoreType.DMA,
            pltpu.SemaphoreType.DMA,
        ],
    )
    def k(dst_hbm, w_hbm, out_hbm, dstv, wv, rows0, rows1, zbuf, deg_s,
          ssem0, ssem1):
        cid = lax.axis_index("c")
        sid = lax.axis_index("s")
        wid = cid * _NS + sid
        pltpu.sync_copy(dst_hbm.at[wid], dstv)
        pltpu.sync_copy(w_hbm.at[wid], wv)

        zv = jnp.zeros((16,), jnp.float32)
        for r in range(16):
            for q in range(8):
                zbuf[r, pl.ds(q * 16, 16)] = zv
        def zrows(e, carry):
            for q in range(1, 8):
                rows0[e, pl.ds(q * 16, 16)] = zv
                rows1[e, pl.ds(q * 16, 16)] = zv
            return carry
        lax.fori_loop(0, _CHUNK, zrows, 0)
        nz = jnp.where(sid == _NS - 1, last // 16, base // 16)

        def zero_chunk(i, carry):
            pltpu.sync_copy(zbuf, deg_s.at[pl.ds(sid * base + i * 16, 16)])
            return carry
        lax.fori_loop(0, nz, zero_chunk, 0)
        plsc.subcore_barrier()

        bufs = ((rows0, ssem0), (rows1, ssem1))

        def fill(j, rows):
            def fill_g(g, c2):
                wvec = wv[j, pl.ds(g * 16, 16)]
                for l in range(16):
                    rows[g * 16 + l, pl.ds(0, 16)] = jnp.full(
                        (16,), wvec[l], jnp.float32)
                return c2
            lax.fori_loop(0, _CHUNK // 16, fill_g, 0)

        fill(0, rows0)

        def chunk_body(j, carry):
            for b in range(2):
                rows, ssem = bufs[b]
                nrows, nssem = bufs[1 - b]

                @pl.when(j % 2 == b)
                def _():
                    @pl.when(j >= 1)
                    def _():
                        pltpu.make_async_copy(
                            nrows, deg_s.at[dstv.at[j - 1]], nssem).wait()
                    pltpu.async_copy(rows, deg_s.at[dstv.at[j]], ssem,
                                     add=True)

                    @pl.when(j + 1 < nchunk)
                    def _():
                        fill(j + 1, nrows)
            return carry
        lax.fori_loop(0, nchunk, chunk_body, 0)
        last_b = (nchunk - 1) % 2
        pltpu.make_async_copy(bufs[last_b][0],
                              deg_s.at[dstv.at[nchunk - 1]],
                              bufs[last_b][1]).wait()
        plsc.subcore_barrier()

        @pl.when(sid < _NS - 1)
        def _():
            pltpu.sync_copy(deg_s.at[pl.ds(sid * base, base)],
                            out_hbm.at[cid, pl.ds(sid * base, base)])

        @pl.when(sid == _NS - 1)
        def _():
            pltpu.sync_copy(deg_s.at[pl.ds((_NS - 1) * base, last)],
                            out_hbm.at[cid, pl.ds((_NS - 1) * base, last)])

    return k(dst3, w3)


def _disy_tc(degp, xw, n_nodes, hid, bnb):
    nb = n_nodes // bnb

    def body(dp_ref, xw_ref, dis_ref, y_ref):
        dp = dp_ref[:, :, :16]
        deg = jnp.sum(dp, axis=(0, 2)) * (1.0 / 16.0) + 1.0
        dis = jnp.where(deg > 0, lax.rsqrt(deg), 0.0)
        dis_ref[...] = dis[:, None]
        y_ref[:, :hid] = xw_ref[...] * dis[:, None]
        y_ref[:, hid:] = jnp.zeros((y_ref.shape[0], 128 - hid), jnp.float32)

    return pl.pallas_call(
        body,
        grid=(nb,),
        in_specs=[
            pl.BlockSpec((_NC, bnb, 128), lambda i: (0, i, 0)),
            pl.BlockSpec((bnb, hid), lambda i: (i, 0)),
        ],
        out_specs=[
            pl.BlockSpec((bnb, 1), lambda i: (i, 0)),
            pl.BlockSpec((bnb, 128), lambda i: (i, 0)),
        ],
        out_shape=[
            jax.ShapeDtypeStruct((n_nodes, 1), jnp.float32),
            jax.ShapeDtypeStruct((n_nodes, 128), jnp.float32),
        ],
    )(degp, xw)


def _sc_msg(src3, dst3, w3, y, n_nodes, hid):
    nchunk = src3.shape[1]
    base = (n_nodes // (8 * _NS)) * 8
    last = n_nodes - (_NS - 1) * base
    mesh = plsc.VectorSubcoreMesh(
        core_axis_name="c", subcore_axis_name="s",
        num_cores=_NC, num_subcores=_NS)

    @functools.partial(
        pl.kernel,
        out_type=pltpu.HBM((_NC, n_nodes, 128), jnp.float32),
        mesh=mesh,
        scratch_types=[
            pltpu.VMEM((nchunk, _CHUNK), jnp.int32),
            pltpu.VMEM((nchunk, _CHUNK), jnp.int32),
            pltpu.VMEM((nchunk, _CHUNK), jnp.float32),
            pltpu.VMEM((_CHUNK, 128), jnp.float32),
            pltpu.VMEM((_CHUNK, 128), jnp.float32),
            pltpu.VMEM((16, 128), jnp.float32),
            pltpu.VMEM_SHARED((n_nodes, 128), jnp.float32),
            pltpu.SemaphoreType.DMA,
            pltpu.SemaphoreType.DMA,
            pltpu.SemaphoreType.DMA,
            pltpu.SemaphoreType.DMA,
        ],
    )
    def k(src_hbm, dst_hbm, w_hbm, y_hbm, out_hbm,
          srcv, dstv, wv, rows0, rows1, zbuf, acc_s,
          gsem0, gsem1, ssem0, ssem1):
        cid = lax.axis_index("c")
        sid = lax.axis_index("s")
        wid = cid * _NS + sid
        pltpu.sync_copy(src_hbm.at[wid], srcv)
        pltpu.sync_copy(dst_hbm.at[wid], dstv)
        pltpu.sync_copy(w_hbm.at[wid], wv)

        zv = jnp.zeros((16,), jnp.float32)
        for r in range(16):
            for q in range(8):
                zbuf[r, pl.ds(q * 16, 16)] = zv
        nz = jnp.where(sid == _NS - 1, last // 16, base // 16)

        def zero_chunk(i, carry):
            pltpu.sync_copy(zbuf, acc_s.at[pl.ds(sid * base + i * 16, 16)])
            return carry
        lax.fori_loop(0, nz, zero_chunk, 0)
        plsc.subcore_barrier()

        bufs = ((rows0, gsem0, ssem0), (rows1, gsem1, ssem1))

        def gather_start(j, rows, gsem):
            pltpu.async_copy(y_hbm.at[srcv.at[j]], rows, gsem)

        def scale_and_scatter(j, rows, gsem, ssem):
            pltpu.make_async_copy(y_hbm.at[srcv.at[j]], rows, gsem).wait()

            def scale(g, c2):
                wvec = wv[j, pl.ds(g * 16, 16)]
                for l in range(16):
                    e = g * 16 + l
                    w = wvec[l]
                    rows[e, pl.ds(0, 16)] = rows[e, pl.ds(0, 16)] * w
                    rows[e, pl.ds(16, 16)] = rows[e, pl.ds(16, 16)] * w
                return c2
            lax.fori_loop(0, _CHUNK // 16, scale, 0)
            pltpu.async_copy(rows, acc_s.at[dstv.at[j]], ssem, add=True)

        gather_start(0, rows0, gsem0)

        def chunk_body(j, carry):
            for b in range(2):
                rows, gsem, ssem = bufs[b]
                nrows, ngsem, nssem = bufs[1 - b]

                @pl.when(j % 2 == b)
                def _():
                    @pl.when(j >= 1)
                    def _():
                        pltpu.make_async_copy(
                            nrows, acc_s.at[dstv.at[j - 1]], nssem).wait()

                    @pl.when(j + 1 < nchunk)
                    def _():
                        gather_start(j + 1, nrows, ngsem)
                    scale_and_scatter(j, rows, gsem, ssem)
            return carry
        lax.fori_loop(0, nchunk, chunk_body, 0)
        last_b = (nchunk - 1) % 2
        pltpu.make_async_copy(bufs[last_b][0],
                              acc_s.at[dstv.at[nchunk - 1]],
                              bufs[last_b][2]).wait()
        plsc.subcore_barrier()

        @pl.when(sid < _NS - 1)
        def _():
            pltpu.sync_copy(acc_s.at[pl.ds(sid * base, base)],
                            out_hbm.at[cid, pl.ds(sid * base, base)])

        @pl.when(sid == _NS - 1)
        def _():
            pltpu.sync_copy(acc_s.at[pl.ds((_NS - 1) * base, last)],
                            out_hbm.at[cid, pl.ds((_NS - 1) * base, last)])

    return k(src3, dst3, w3, y)


def _head_tc(accp, dis, xw, h, gcnb2, wg, wh, ob2, n_nodes, hid, bnb):
    nb = n_nodes // bnb

    def body(a_ref, dis_ref, xw_ref, h_ref, gcnb_ref, wg_ref, wh_ref, ob_ref,
             o_ref):
        dis_c = dis_ref[...]
        acc = a_ref[0, :, :hid] + a_ref[1, :, :hid]
        gcn = (acc * dis_c
               + xw_ref[...] * (dis_c * dis_c) + gcnb_ref[...])
        zg = jnp.where(gcn > 0, gcn, jnp.exp(jnp.minimum(gcn, 0.0)) - 1.0)
        hh = h_ref[...]
        zh = jnp.where(hh > 0, hh, jnp.exp(jnp.minimum(hh, 0.0)) - 1.0)
        s = (jnp.dot(zg, wg_ref[...], preferred_element_type=jnp.float32)
             + jnp.dot(zh, wh_ref[...], preferred_element_type=jnp.float32)
             + ob_ref[...])
        o_ref[...] = jax.nn.sigmoid(s)

    return pl.pallas_call(
        body,
        grid=(nb,),
        in_specs=[
            pl.BlockSpec((_NC, bnb, 128), lambda i: (0, i, 0)),
            pl.BlockSpec((bnb, 1), lambda i: (i, 0)),
            pl.BlockSpec((bnb, hid), lambda i: (i, 0)),
            pl.BlockSpec((bnb, hid), lambda i: (i, 0)),
            pl.BlockSpec((1, hid), lambda i: (0, 0)),
            pl.BlockSpec((hid, 1), lambda i: (0, 0)),
            pl.BlockSpec((hid, 1), lambda i: (0, 0)),
            pl.BlockSpec((1, 1), lambda i: (0, 0)),
        ],
        out_specs=pl.BlockSpec((bnb, 1), lambda i: (i, 0)),
        out_shape=jax.ShapeDtypeStruct((n_nodes, 1), jnp.float32),
    )(accp, dis, xw, h, gcnb2, wg, wh, ob2)


def kernel(x, edge_index, edge_attr, W_ih, W_hh, b_ih, b_hh,
           gcn_W, gcn_b, out_W, out_b):
    n_nodes, t_steps, d_in = x.shape
    hid = W_hh.shape[1]
    e_edges = edge_attr.shape[0]
    nw = _NC * _NS

    grp = nw * _CHUNK
    epad = ((e_edges + grp - 1) // grp) * grp
    pad = epad - e_edges
    src = edge_index[0].astype(jnp.int32)
    dst = edge_index[1].astype(jnp.int32)
    w = edge_attr
    if pad:
        src = jnp.concatenate([src, jnp.zeros((pad,), jnp.int32)])
        dst = jnp.concatenate([dst, jnp.zeros((pad,), jnp.int32)])
        w = jnp.concatenate([w, jnp.zeros((pad,), w.dtype)])
    nchunk = epad // grp
    src3 = src.reshape(nw, nchunk, _CHUNK)
    dst3 = dst.reshape(nw, nchunk, _CHUNK)
    w3 = w.reshape(nw, nchunk, _CHUNK)

    wihT = W_ih.T
    whhT = jnp.concatenate(
        [W_hh.T, jnp.zeros((128 - hid, 4 * hid), jnp.float32)], axis=0)
    b2 = (b_ih + b_hh).reshape(1, 4 * hid)
    gcnwT = gcn_W.T
    gcnb2 = gcn_b.reshape(1, hid)
    wg = out_W[:, :hid].T
    wh = out_W[:, hid:].T
    ob2 = out_b.reshape(1, 1)

    xt3 = jnp.swapaxes(x, 0, 1)
    h, xw = _lstm_tc(xt3, wihT, whhT, b2, gcnwT,
                     n_nodes, t_steps, d_in, hid, bn=1000)
    degp = _sc_deg(dst3, w3, n_nodes)
    dis, y = _disy_tc(degp, xw, n_nodes, hid, bnb=2000)
    accp = _sc_msg(src3, dst3, w3, y, n_nodes, hid)
    out = _head_tc(accp, dis, xw, h, gcnb2, wg, wh, ob2,
                   n_nodes, hid, bnb=2000)
    return (out, 0)


# speedup vs baseline: 13.4281x; 1.8063x over previous
import functools

import jax
import jax.numpy as jnp
from jax import lax
from jax.experimental import pallas as pl
from jax.experimental.pallas import tpu as pltpu
from jax.experimental.pallas import tpu_sc as plsc

_NC = 2
_NS = 16
_CHUNK = 128


def _lstm_tc(x2, wihT, whhT, b2, gcnwT, n_nodes, t_steps, d_in, hid, bn):
    nb = n_nodes // bn

    def body(x_ref, wih_ref, whh_ref, b_ref, gcnw_ref, h_ref, xw_ref):
        hp = jnp.zeros((bn, 128), jnp.float32)
        c = jnp.zeros((bn, hid), jnp.float32)
        zpad = jnp.zeros((bn, 128 - hid), jnp.float32)
        wih = wih_ref[...]
        whh = whh_ref[...]
        b = b_ref[...]
        h = None
        for t in range(t_steps):
            xt = x_ref[t]
            gates = (jnp.dot(xt, wih, preferred_element_type=jnp.float32)
                     + jnp.dot(hp, whh, preferred_element_type=jnp.float32)
                     + b)
            i = jax.nn.sigmoid(gates[:, :hid])
            f = jax.nn.sigmoid(gates[:, hid:2 * hid])
            g = jnp.tanh(gates[:, 2 * hid:3 * hid])
            o = jax.nn.sigmoid(gates[:, 3 * hid:])
            c = f * c + i * g
            h = o * jnp.tanh(c)
            hp = jnp.concatenate([h, zpad], axis=1)
        h_ref[...] = h
        xw_ref[...] = jnp.dot(h, gcnw_ref[...], preferred_element_type=jnp.float32)

    return pl.pallas_call(
        body,
        grid=(nb,),
        in_specs=[
            pl.BlockSpec((t_steps, bn, d_in), lambda i: (0, i, 0)),
            pl.BlockSpec((d_in, 4 * hid), lambda i: (0, 0)),
            pl.BlockSpec((128, 4 * hid), lambda i: (0, 0)),
            pl.BlockSpec((1, 4 * hid), lambda i: (0, 0)),
            pl.BlockSpec((hid, hid), lambda i: (0, 0)),
        ],
        out_specs=[
            pl.BlockSpec((bn, hid), lambda i: (i, 0)),
            pl.BlockSpec((bn, hid), lambda i: (i, 0)),
        ],
        out_shape=[
            jax.ShapeDtypeStruct((n_nodes, hid), jnp.float32),
            jax.ShapeDtypeStruct((n_nodes, hid), jnp.float32),
        ],
        compiler_params=pltpu.CompilerParams(
            dimension_semantics=("arbitrary",)),
    )(x2, wihT, whhT, b2, gcnwT)


def _sc_deg(dst3, w3, n_nodes):
    nchunk = dst3.shape[1]
    base = (n_nodes // (8 * _NS)) * 8
    last = n_nodes - (_NS - 1) * base
    mesh = plsc.VectorSubcoreMesh(
        core_axis_name="c", subcore_axis_name="s",
        num_cores=_NC, num_subcores=_NS)

    @functools.partial(
        pl.kernel,
        out_type=pltpu.HBM((_NC, n_nodes, 128), jnp.float32),
        mesh=mesh,
        scratch_types=[
            pltpu.VMEM((nchunk, _CHUNK), jnp.int32),
            pltpu.VMEM((nchunk, _CHUNK), jnp.float32),
            pltpu.VMEM((_CHUNK, 128), jnp.float32),
            pltpu.VMEM((_CHUNK, 128), jnp.float32),
            pltpu.VMEM((16, 128), jnp.float32),
            pltpu.VMEM_SHARED((n_nodes, 128), jnp.float32),
            pltpu.SemaphoreType.DMA,
            pltpu.SemaphoreType.DMA,
        ],
    )
    def k(dst_hbm, w_hbm, out_hbm, dstv, wv, rows0, rows1, zbuf, deg_s,
          ssem0, ssem1):
        cid = lax.axis_index("c")
        sid = lax.axis_index("s")
        wid = cid * _NS + sid
        pltpu.sync_copy(dst_hbm.at[wid], dstv)
        pltpu.sync_copy(w_hbm.at[wid], wv)

        zv = jnp.zeros((16,), jnp.float32)
        for r in range(16):
            for q in range(8):
                zbuf[r, pl.ds(q * 16, 16)] = zv
        def zrows(e, carry):
            for q in range(1, 8):
                rows0[e, pl.ds(q * 16, 16)] = zv
                rows1[e, pl.ds(q * 16, 16)] = zv
            return carry
        lax.fori_loop(0, _CHUNK, zrows, 0)
        nz = jnp.where(sid == _NS - 1, last // 16, base // 16)

        def zero_chunk(i, carry):
            pltpu.sync_copy(zbuf, deg_s.at[pl.ds(sid * base + i * 16, 16)])
            return carry
        lax.fori_loop(0, nz, zero_chunk, 0)
        plsc.subcore_barrier()

        bufs = ((rows0, ssem0), (rows1, ssem1))

        def fill(j, rows):
            def fill_g(g, c2):
                wvec = wv[j, pl.ds(g * 16, 16)]
                for l in range(16):
                    rows[g * 16 + l, pl.ds(0, 16)] = jnp.full(
                        (16,), wvec[l], jnp.float32)
                return c2
            lax.fori_loop(0, _CHUNK // 16, fill_g, 0)

        fill(0, rows0)

        def chunk_body(j, carry):
            for b in range(2):
                rows, ssem = bufs[b]
                nrows, nssem = bufs[1 - b]

                @pl.when(j % 2 == b)
                def _():
                    @pl.when(j >= 1)
                    def _():
                        pltpu.make_async_copy(
                            nrows, deg_s.at[dstv.at[j - 1]], nssem).wait()
                    pltpu.async_copy(rows, deg_s.at[dstv.at[j]], ssem,
                                     add=True)

                    @pl.when(j + 1 < nchunk)
                    def _():
                        fill(j + 1, nrows)
            return carry
        lax.fori_loop(0, nchunk, chunk_body, 0)
        last_b = (nchunk - 1) % 2
        pltpu.make_async_copy(bufs[last_b][0],
                              deg_s.at[dstv.at[nchunk - 1]],
                              bufs[last_b][1]).wait()
        plsc.subcore_barrier()

        @pl.when(sid < _NS - 1)
        def _():
            pltpu.sync_copy(deg_s.at[pl.ds(sid * base, base)],
                            out_hbm.at[cid, pl.ds(sid * base, base)])

        @pl.when(sid == _NS - 1)
        def _():
            pltpu.sync_copy(deg_s.at[pl.ds((_NS - 1) * base, last)],
                            out_hbm.at[cid, pl.ds((_NS - 1) * base, last)])

    return k(dst3, w3)


def _disy_tc(degp, xw, n_nodes, hid, bnb):
    nb = n_nodes // bnb

    def body(dp_ref, xw_ref, dis_ref, y_ref):
        dp = dp_ref[:, :, :16]
        deg = jnp.sum(dp, axis=(0, 2)) * (1.0 / 16.0) + 1.0
        dis = jnp.where(deg > 0, lax.rsqrt(deg), 0.0)
        dis_ref[...] = dis[:, None]
        y_ref[:, :hid] = xw_ref[...] * dis[:, None]
        y_ref[:, hid:] = jnp.zeros((y_ref.shape[0], 128 - hid), jnp.float32)

    return pl.pallas_call(
        body,
        grid=(nb,),
        in_specs=[
            pl.BlockSpec((_NC, bnb, 128), lambda i: (0, i, 0)),
            pl.BlockSpec((bnb, hid), lambda i: (i, 0)),
        ],
        out_specs=[
            pl.BlockSpec((bnb, 1), lambda i: (i, 0)),
            pl.BlockSpec((bnb, 128), lambda i: (i, 0)),
        ],
        out_shape=[
            jax.ShapeDtypeStruct((n_nodes, 1), jnp.float32),
            jax.ShapeDtypeStruct((n_nodes, 128), jnp.float32),
        ],
    )(degp, xw)


def _sc_msg(src3, dst3, w3, y, n_nodes, hid):
    nchunk = src3.shape[1]
    base = (n_nodes // (8 * _NS)) * 8
    last = n_nodes - (_NS - 1) * base
    mesh = plsc.VectorSubcoreMesh(
        core_axis_name="c", subcore_axis_name="s",
        num_cores=_NC, num_subcores=_NS)

    @functools.partial(
        pl.kernel,
        out_type=pltpu.HBM((_NC, n_nodes, 128), jnp.float32),
        mesh=mesh,
        scratch_types=[
            pltpu.VMEM((nchunk, _CHUNK), jnp.int32),
            pltpu.VMEM((nchunk, _CHUNK), jnp.int32),
            pltpu.VMEM((nchunk, _CHUNK), jnp.float32),
            pltpu.VMEM((_CHUNK, 128), jnp.float32),
            pltpu.VMEM((_CHUNK, 128), jnp.float32),
            pltpu.VMEM((16, 128), jnp.float32),
            pltpu.VMEM_SHARED((n_nodes, 128), jnp.float32),
            pltpu.SemaphoreType.DMA,
            pltpu.SemaphoreType.DMA,
            pltpu.SemaphoreType.DMA,
            pltpu.SemaphoreType.DMA,
        ],
    )
    def k(src_hbm, dst_hbm, w_hbm, y_hbm, out_hbm,
          srcv, dstv, wv, rows0, rows1, zbuf, acc_s,
          gsem0, gsem1, ssem0, ssem1):
        cid = lax.axis_index("c")
        sid = lax.axis_index("s")
        wid = cid * _NS + sid
        pltpu.sync_copy(src_hbm.at[wid], srcv)
        pltpu.sync_copy(dst_hbm.at[wid], dstv)
        pltpu.sync_copy(w_hbm.at[wid], wv)

        zv = jnp.zeros((16,), jnp.float32)
        for r in range(16):
            for q in range(8):
                zbuf[r, pl.ds(q * 16, 16)] = zv
        nz = jnp.where(sid == _NS - 1, last // 16, base // 16)

        def zero_chunk(i, carry):
            pltpu.sync_copy(zbuf, acc_s.at[pl.ds(sid * base + i * 16, 16)])
            return carry
        lax.fori_loop(0, nz, zero_chunk, 0)
        plsc.subcore_barrier()

        bufs = ((rows0, gsem0, ssem0), (rows1, gsem1, ssem1))

        def gather_start(j, rows, gsem):
            pltpu.async_copy(y_hbm.at[srcv.at[j]], rows, gsem)

        def scale_and_scatter(j, rows, gsem, ssem):
            pltpu.make_async_copy(y_hbm.at[srcv.at[j]], rows, gsem).wait()

            def scale(g, c2):
                wvec = wv[j, pl.ds(g * 16, 16)]
                for l in range(16):
                    e = g * 16 + l
                    w = wvec[l]
                    rows[e, pl.ds(0, 16)] = rows[e, pl.ds(0, 16)] * w
                    rows[e, pl.ds(16, 16)] = rows[e, pl.ds(16, 16)] * w
                return c2
            lax.fori_loop(0, _CHUNK // 16, scale, 0)
            pltpu.async_copy(rows, acc_s.at[dstv.at[j]], ssem, add=True)

        gather_start(0, rows0, gsem0)

        def chunk_body(j, carry):
            for b in range(2):
                rows, gsem, ssem = bufs[b]
                nrows, ngsem, nssem = bufs[1 - b]

                @pl.when(j % 2 == b)
                def _():
                    @pl.when(j >= 1)
                    def _():
                        pltpu.make_async_copy(
                            nrows, acc_s.at[dstv.at[j - 1]], nssem).wait()

                    @pl.when(j + 1 < nchunk)
                    def _():
                        gather_start(j + 1, nrows, ngsem)
                    scale_and_scatter(j, rows, gsem, ssem)
            return carry
        lax.fori_loop(0, nchunk, chunk_body, 0)
        last_b = (nchunk - 1) % 2
        pltpu.make_async_copy(bufs[last_b][0],
                              acc_s.at[dstv.at[nchunk - 1]],
                              bufs[last_b][2]).wait()
        plsc.subcore_barrier()

        @pl.when(sid < _NS - 1)
        def _():
            pltpu.sync_copy(acc_s.at[pl.ds(sid * base, base)],
                            out_hbm.at[cid, pl.ds(sid * base, base)])

        @pl.when(sid == _NS - 1)
        def _():
            pltpu.sync_copy(acc_s.at[pl.ds((_NS - 1) * base, last)],
                            out_hbm.at[cid, pl.ds((_NS - 1) * base, last)])

    return k(src3, dst3, w3, y)


def _head_tc(accp, dis, xw, h, gcnb2, wg, wh, ob2, n_nodes, hid, bnb):
    nb = n_nodes // bnb

    def body(a_ref, dis_ref, xw_ref, h_ref, gcnb_ref, wg_ref, wh_ref, ob_ref,
             o_ref):
        dis_c = dis_ref[...]
        acc = a_ref[0, :, :hid] + a_ref[1, :, :hid]
        gcn = (acc * dis_c
               + xw_ref[...] * (dis_c * dis_c) + gcnb_ref[...])
        zg = jnp.where(gcn > 0, gcn, jnp.exp(jnp.minimum(gcn, 0.0)) - 1.0)
        hh = h_ref[...]
        zh = jnp.where(hh > 0, hh, jnp.exp(jnp.minimum(hh, 0.0)) - 1.0)
        s = (jnp.dot(zg, wg_ref[...], preferred_element_type=jnp.float32)
             + jnp.dot(zh, wh_ref[...], preferred_element_type=jnp.float32)
             + ob_ref[...])
        o_ref[...] = jax.nn.sigmoid(s)

    return pl.pallas_call(
        body,
        grid=(nb,),
        in_specs=[
            pl.BlockSpec((_NC, bnb, 128), lambda i: (0, i, 0)),
            pl.BlockSpec((bnb, 1), lambda i: (i, 0)),
            pl.BlockSpec((bnb, hid), lambda i: (i, 0)),
            pl.BlockSpec((bnb, hid), lambda i: (i, 0)),
            pl.BlockSpec((1, hid), lambda i: (0, 0)),
            pl.BlockSpec((hid, 1), lambda i: (0, 0)),
            pl.BlockSpec((hid, 1), lambda i: (0, 0)),
            pl.BlockSpec((1, 1), lambda i: (0, 0)),
        ],
        out_specs=pl.BlockSpec((bnb, 1), lambda i: (i, 0)),
        out_shape=jax.ShapeDtypeStruct((n_nodes, 1), jnp.float32),
    )(accp, dis, xw, h, gcnb2, wg, wh, ob2)


def kernel(x, edge_index, edge_attr, W_ih, W_hh, b_ih, b_hh,
           gcn_W, gcn_b, out_W, out_b):
    n_nodes, t_steps, d_in = x.shape
    hid = W_hh.shape[1]
    e_edges = edge_attr.shape[0]
    nw = _NC * _NS

    grp = nw * _CHUNK
    epad = ((e_edges + grp - 1) // grp) * grp
    pad = epad - e_edges
    src = edge_index[0].astype(jnp.int32)
    dst = edge_index[1].astype(jnp.int32)
    w = edge_attr
    if pad:
        spread = jnp.arange(pad, dtype=jnp.int32) % jnp.int32(n_nodes)
        src = jnp.concatenate([src, spread])
        dst = jnp.concatenate([dst, spread])
        w = jnp.concatenate([w, jnp.zeros((pad,), w.dtype)])
    nchunk = epad // grp
    src3 = src.reshape(nw, nchunk, _CHUNK)
    dst3 = dst.reshape(nw, nchunk, _CHUNK)
    w3 = w.reshape(nw, nchunk, _CHUNK)

    wihT = W_ih.T
    whhT = jnp.concatenate(
        [W_hh.T, jnp.zeros((128 - hid, 4 * hid), jnp.float32)], axis=0)
    b2 = (b_ih + b_hh).reshape(1, 4 * hid)
    gcnwT = gcn_W.T
    gcnb2 = gcn_b.reshape(1, hid)
    wg = out_W[:, :hid].T
    wh = out_W[:, hid:].T
    ob2 = out_b.reshape(1, 1)

    xt3 = jnp.swapaxes(x, 0, 1)
    h, xw = _lstm_tc(xt3, wihT, whhT, b2, gcnwT,
                     n_nodes, t_steps, d_in, hid, bn=1000)
    degp = _sc_deg(dst3, w3, n_nodes)
    dis, y = _disy_tc(degp, xw, n_nodes, hid, bnb=2000)
    accp = _sc_msg(src3, dst3, w3, y, n_nodes, hid)
    out = _head_tc(accp, dis, xw, h, gcnb2, wg, wh, ob2,
                   n_nodes, hid, bnb=2000)
    return (out, 0)
